# baseline (speedup 1.0000x reference)
import functools

import jax
import jax.numpy as jnp
from jax import lax
from jax.experimental import pallas as pl
from jax.experimental.pallas import tpu as pltpu
from jax.experimental.pallas import tpu_sc as plsc

D_MODEL = 64
SCALE = 8.0

NUM_CORES = 2
NUM_SUBCORES = 16
NUM_WORKERS = NUM_CORES * NUM_SUBCORES

SEQ = 200
BATCH_TILES = 32
UNIT = 128
UNITS_TOTAL = SEQ * BATCH_TILES
UNITS_PER_W = UNITS_TOTAL // NUM_WORKERS
LANES = 16
NBUF = 4
PAD_UNIT = UNIT + 2


def _embed(xt_flat, table):
    mesh = plsc.VectorSubcoreMesh(core_axis_name="c", subcore_axis_name="s")

    @functools.partial(
        pl.kernel,
        mesh=mesh,
        out_type=jax.ShapeDtypeStruct(
            (SEQ, D_MODEL // 8, BATCH_TILES, 8, UNIT), jnp.float32
        ),
        compiler_params=pltpu.CompilerParams(
            use_tc_tiling_on_sc=False, needs_layout_passes=False
        ),
        scratch_types=(
            [pltpu.VMEM((UNITS_PER_W * UNIT,), jnp.int32)]
            + [pltpu.VMEM((UNIT, D_MODEL), jnp.float32) for _ in range(NBUF)]
            + [pltpu.VMEM((D_MODEL // 8, 8, PAD_UNIT), jnp.float32)
               for _ in range(NBUF)]
            + [pltpu.SemaphoreType.DMA for _ in range(2 * NBUF)]
        ),
    )
    def body(x_hbm, table_hbm, out_hbm, idx_v, *bufs):
        rows_b = bufs[0:NBUF]
        trans_b = bufs[NBUF:2 * NBUF]
        gsems = bufs[2 * NBUF:3 * NBUF]
        wsems = bufs[3 * NBUF:4 * NBUF]

        wid = lax.axis_index("s") * NUM_CORES + lax.axis_index("c")
        u_base = wid * UNITS_PER_W

        pltpu.sync_copy(
            x_hbm.at[pl.ds(u_base * UNIT, UNITS_PER_W * UNIT)], idx_v
        )

        def fire_gather(slot, u_rel):
            pltpu.async_copy(
                table_hbm.at[idx_v.at[pl.ds(u_rel * UNIT, UNIT)]],
                rows_b[slot],
                gsems[slot],
            )

        for slot in range(NBUF):
            fire_gather(slot, slot)

        iota = lax.iota(jnp.int32, LANES)
        dt_idx = [iota // 8 + dc * 2 for dc in range(D_MODEL // LANES)]
        dr_idx = iota % 8

        def substep(slot, i, u_rel):
            u = u_base + u_rel
            s = u // BATCH_TILES
            bt = u % BATCH_TILES

            pltpu.make_async_copy(
                table_hbm.at[idx_v.at[pl.ds(0, UNIT)]],
                rows_b[slot],
                gsems[slot],
            ).wait()

            u_prev = u - NBUF
            s_prev = u_prev // BATCH_TILES
            bt_prev = u_prev % BATCH_TILES

            @pl.when(i > 0)
            def _():
                pltpu.make_async_copy(
                    trans_b[slot].at[:, :, pl.ds(0, UNIT)],
                    out_hbm.at[s_prev, :, bt_prev],
                    wsems[slot],
                ).wait()

            @plsc.parallel_loop(0, UNIT, 1, unroll=8)
            def tr_body(j):
                col = jnp.full((LANES,), j, dtype=jnp.int32)
                for dc in range(D_MODEL // LANES):
                    v = rows_b[slot][j, pl.ds(dc * LANES, LANES)]
                    plsc.store_scatter(
                        trans_b[slot], [dt_idx[dc], dr_idx, col], v
                    )

            pltpu.async_copy(
                trans_b[slot].at[:, :, pl.ds(0, UNIT)],
                out_hbm.at[s, :, bt],
                wsems[slot],
            )

            @pl.when(u_rel + NBUF < UNITS_PER_W)
            def _():
                fire_gather(slot, u_rel + NBUF)

        def loop_body(i, carry):
            for slot in range(NBUF):
                substep(slot, i, NBUF * i + slot)
            return carry

        lax.fori_loop(0, UNITS_PER_W // NBUF, loop_body, 0)

        for k in range(NBUF):
            u_rel = UNITS_PER_W - NBUF + k
            slot = u_rel % NBUF
            u = u_base + u_rel
            s = u // BATCH_TILES
            bt = u % BATCH_TILES
            pltpu.make_async_copy(
                trans_b[slot].at[:, :, pl.ds(0, UNIT)],
                out_hbm.at[s, :, bt],
                wsems[slot],
            ).wait()

    return body(xt_flat, table)


VOCAB = 1000000
TCOLS = 2048


def _detile_table(table_t):
    def tk(in_ref, out_ref):
        x = in_ref[...]
        eye_scaled = jnp.eye(D_MODEL, dtype=jnp.float32) * SCALE
        xt = lax.dot_general(
            x, eye_scaled,
            dimension_numbers=(((0,), (0,)), ((), ())),
            preferred_element_type=jnp.float32,
        )
        y = xt.reshape(TCOLS // 2, 2, D_MODEL)
        out_ref[:, 0:D_MODEL] = y[:, 0, :]
        out_ref[:, D_MODEL:2 * D_MODEL] = y[:, 1, :]

    return pl.pallas_call(
        tk,
        grid=((VOCAB + TCOLS - 1) // TCOLS,),
        in_specs=[pl.BlockSpec((D_MODEL, TCOLS), lambda i: (0, i))],
        out_specs=pl.BlockSpec((TCOLS // 2, 2 * D_MODEL), lambda i: (i, 0)),
        out_shape=jax.ShapeDtypeStruct((VOCAB // 2, 2 * D_MODEL), jnp.float32),
    )(table_t)


def kernel(x, table):
    batch, seq = x.shape
    xt_flat = x.T.reshape(batch * seq)
    table_lin = _detile_table(table.T).reshape(VOCAB, D_MODEL)
    out5 = _embed(xt_flat, table_lin)
    out = out5.transpose(2, 4, 0, 1, 3).reshape(batch, seq, D_MODEL)
    return out


# speedup vs baseline: 1.2924x; 1.2924x over previous
import functools

import jax
import jax.numpy as jnp
from jax import lax
from jax.experimental import pallas as pl
from jax.experimental.pallas import tpu as pltpu
from jax.experimental.pallas import tpu_sc as plsc

D_MODEL = 64
SCALE = 8.0

NUM_CORES = 2
NUM_SUBCORES = 16
NUM_WORKERS = NUM_CORES * NUM_SUBCORES

SEQ = 200
BATCH_TILES = 32
UNIT = 128
UNITS_TOTAL = SEQ * BATCH_TILES
UNITS_PER_W = UNITS_TOTAL // NUM_WORKERS
LANES = 16
NBUF = 4
PAD_UNIT = UNIT + 2


def _embed(xt_flat, table):
    mesh = plsc.VectorSubcoreMesh(core_axis_name="c", subcore_axis_name="s")

    @functools.partial(
        pl.kernel,
        mesh=mesh,
        out_type=jax.ShapeDtypeStruct(
            (SEQ, D_MODEL // 8, BATCH_TILES, 8, UNIT), jnp.float32
        ),
        compiler_params=pltpu.CompilerParams(
            use_tc_tiling_on_sc=False, needs_layout_passes=False
        ),
        scratch_types=(
            [pltpu.VMEM((UNITS_PER_W * UNIT,), jnp.int32)]
            + [pltpu.VMEM((UNIT, D_MODEL), jnp.float32) for _ in range(NBUF)]
            + [pltpu.VMEM((D_MODEL // 8, 8, PAD_UNIT), jnp.float32)
               for _ in range(NBUF)]
            + [pltpu.SemaphoreType.DMA for _ in range(2 * NBUF)]
        ),
    )
    def body(x_hbm, table_hbm, out_hbm, idx_v, *bufs):
        rows_b = bufs[0:NBUF]
        trans_b = bufs[NBUF:2 * NBUF]
        gsems = bufs[2 * NBUF:3 * NBUF]
        wsems = bufs[3 * NBUF:4 * NBUF]

        wid = lax.axis_index("s") * NUM_CORES + lax.axis_index("c")
        u_base = wid * UNITS_PER_W

        pltpu.sync_copy(
            x_hbm.at[pl.ds(u_base * UNIT, UNITS_PER_W * UNIT)], idx_v
        )

        def fire_gather(slot, u_rel):
            pltpu.async_copy(
                table_hbm.at[idx_v.at[pl.ds(u_rel * UNIT, UNIT)]],
                rows_b[slot],
                gsems[slot],
            )

        for slot in range(NBUF):
            fire_gather(slot, slot)

        iota = lax.iota(jnp.int32, LANES)
        dt_idx = [iota // 8 + dc * 2 for dc in range(D_MODEL // LANES)]
        dr_idx = iota % 8

        def substep(slot, i, u_rel):
            u = u_base + u_rel
            s = u // BATCH_TILES
            bt = u % BATCH_TILES

            pltpu.make_async_copy(
                table_hbm.at[idx_v.at[pl.ds(0, UNIT)]],
                rows_b[slot],
                gsems[slot],
            ).wait()

            u_prev = u - NBUF
            s_prev = u_prev // BATCH_TILES
            bt_prev = u_prev % BATCH_TILES

            @pl.when(i > 0)
            def _():
                pltpu.make_async_copy(
                    trans_b[slot].at[:, :, pl.ds(0, UNIT)],
                    out_hbm.at[s_prev, :, bt_prev],
                    wsems[slot],
                ).wait()

            @plsc.parallel_loop(0, UNIT, 1, unroll=8)
            def tr_body(j):
                col = jnp.full((LANES,), j, dtype=jnp.int32)
                for dc in range(D_MODEL // LANES):
                    v = rows_b[slot][j, pl.ds(dc * LANES, LANES)]
                    plsc.store_scatter(
                        trans_b[slot], [dt_idx[dc], dr_idx, col], v
                    )

            pltpu.async_copy(
                trans_b[slot].at[:, :, pl.ds(0, UNIT)],
                out_hbm.at[s, :, bt],
                wsems[slot],
            )

            @pl.when(u_rel + NBUF < UNITS_PER_W)
            def _():
                fire_gather(slot, u_rel + NBUF)

        def loop_body(i, carry):
            for slot in range(NBUF):
                substep(slot, i, NBUF * i + slot)
            return carry

        lax.fori_loop(0, UNITS_PER_W // NBUF, loop_body, 0)

        for k in range(NBUF):
            u_rel = UNITS_PER_W - NBUF + k
            slot = u_rel % NBUF
            u = u_base + u_rel
            s = u // BATCH_TILES
            bt = u % BATCH_TILES
            pltpu.make_async_copy(
                trans_b[slot].at[:, :, pl.ds(0, UNIT)],
                out_hbm.at[s, :, bt],
                wsems[slot],
            ).wait()

    return body(xt_flat, table)


VOCAB = 1000000
TCOLS = 8192


def _detile_table(table_t):
    def tk(in_ref, out_ref):
        x = in_ref[...]
        xt = x.T * SCALE
        y = xt.reshape(TCOLS // 2, 2, D_MODEL)
        out_ref[:, 0:D_MODEL] = y[:, 0, :]
        out_ref[:, D_MODEL:2 * D_MODEL] = y[:, 1, :]

    return pl.pallas_call(
        tk,
        grid=((VOCAB + TCOLS - 1) // TCOLS,),
        in_specs=[pl.BlockSpec((D_MODEL, TCOLS), lambda i: (0, i))],
        out_specs=pl.BlockSpec((TCOLS // 2, 2 * D_MODEL), lambda i: (i, 0)),
        out_shape=jax.ShapeDtypeStruct((VOCAB // 2, 2 * D_MODEL), jnp.float32),
    )(table_t)


def kernel(x, table):
    batch, seq = x.shape
    xt_flat = x.T.reshape(batch * seq)
    table_lin = _detile_table(table.T).reshape(VOCAB, D_MODEL)
    out5 = _embed(xt_flat, table_lin)
    out = out5.transpose(2, 4, 0, 1, 3).reshape(batch, seq, D_MODEL)
    return out


# speedup vs baseline: 1.3108x; 1.0142x over previous
import functools

import jax
import jax.numpy as jnp
from jax import lax
from jax.experimental import pallas as pl
from jax.experimental.pallas import tpu as pltpu
from jax.experimental.pallas import tpu_sc as plsc

D_MODEL = 64
SCALE = 8.0

NUM_CORES = 2
NUM_SUBCORES = 16
NUM_WORKERS = NUM_CORES * NUM_SUBCORES

SEQ = 200
BATCH_TILES = 32
UNIT = 128
UNITS_TOTAL = SEQ * BATCH_TILES
UNITS_PER_W = UNITS_TOTAL // NUM_WORKERS
LANES = 16
NBUF = 4
PAD_UNIT = UNIT + 2


def _embed(xt_flat, table):
    mesh = plsc.VectorSubcoreMesh(core_axis_name="c", subcore_axis_name="s")

    @functools.partial(
        pl.kernel,
        mesh=mesh,
        out_type=jax.ShapeDtypeStruct(
            (SEQ, D_MODEL // 8, BATCH_TILES, 8, UNIT), jnp.float32
        ),
        compiler_params=pltpu.CompilerParams(
            use_tc_tiling_on_sc=False, needs_layout_passes=False
        ),
        scratch_types=(
            [pltpu.VMEM((UNITS_PER_W * UNIT,), jnp.int32)]
            + [pltpu.VMEM((UNIT, D_MODEL), jnp.float32) for _ in range(NBUF)]
            + [pltpu.VMEM((D_MODEL // 8, 8, PAD_UNIT), jnp.float32)
               for _ in range(NBUF)]
            + [pltpu.SemaphoreType.DMA for _ in range(2 * NBUF)]
        ),
    )
    def body(x_hbm, table_hbm, out_hbm, idx_v, *bufs):
        rows_b = bufs[0:NBUF]
        trans_b = bufs[NBUF:2 * NBUF]
        gsems = bufs[2 * NBUF:3 * NBUF]
        wsems = bufs[3 * NBUF:4 * NBUF]

        wid = lax.axis_index("s") * NUM_CORES + lax.axis_index("c")
        u_base = wid * UNITS_PER_W

        pltpu.sync_copy(
            x_hbm.at[pl.ds(u_base * UNIT, UNITS_PER_W * UNIT)], idx_v
        )

        def fire_gather(slot, u_rel):
            pltpu.async_copy(
                table_hbm.at[idx_v.at[pl.ds(u_rel * UNIT, UNIT)]],
                rows_b[slot],
                gsems[slot],
            )

        for slot in range(NBUF):
            fire_gather(slot, slot)

        iota = lax.iota(jnp.int32, LANES)
        dt_idx = [iota // 8 + dc * 2 for dc in range(D_MODEL // LANES)]
        dr_idx = iota % 8

        def substep(slot, i, u_rel):
            u = u_base + u_rel
            s = u // BATCH_TILES
            bt = u % BATCH_TILES

            pltpu.make_async_copy(
                table_hbm.at[idx_v.at[pl.ds(0, UNIT)]],
                rows_b[slot],
                gsems[slot],
            ).wait()

            u_prev = u - NBUF
            s_prev = u_prev // BATCH_TILES
            bt_prev = u_prev % BATCH_TILES

            @pl.when(i > 0)
            def _():
                pltpu.make_async_copy(
                    trans_b[slot].at[:, :, pl.ds(0, UNIT)],
                    out_hbm.at[s_prev, :, bt_prev],
                    wsems[slot],
                ).wait()

            @plsc.parallel_loop(0, UNIT, 1, unroll=8)
            def tr_body(j):
                col = jnp.full((LANES,), j, dtype=jnp.int32)
                for dc in range(D_MODEL // LANES):
                    v = rows_b[slot][j, pl.ds(dc * LANES, LANES)]
                    plsc.store_scatter(
                        trans_b[slot], [dt_idx[dc], dr_idx, col], v
                    )

            pltpu.async_copy(
                trans_b[slot].at[:, :, pl.ds(0, UNIT)],
                out_hbm.at[s, :, bt],
                wsems[slot],
            )

            @pl.when(u_rel + NBUF < UNITS_PER_W)
            def _():
                fire_gather(slot, u_rel + NBUF)

        def loop_body(i, carry):
            for slot in range(NBUF):
                substep(slot, i, NBUF * i + slot)
            return carry

        lax.fori_loop(0, UNITS_PER_W // NBUF, loop_body, 0)

        for k in range(NBUF):
            u_rel = UNITS_PER_W - NBUF + k
            slot = u_rel % NBUF
            u = u_base + u_rel
            s = u // BATCH_TILES
            bt = u % BATCH_TILES
            pltpu.make_async_copy(
                trans_b[slot].at[:, :, pl.ds(0, UNIT)],
                out_hbm.at[s, :, bt],
                wsems[slot],
            ).wait()

    return body(xt_flat, table)


VOCAB = 1000000
TCOLS = 16384


def _detile_table(table_t):
    def tk(in_ref, out_ref):
        x = in_ref[...]
        xt = x.T * SCALE
        y = xt.reshape(TCOLS // 2, 2, D_MODEL)
        out_ref[:, 0:D_MODEL] = y[:, 0, :]
        out_ref[:, D_MODEL:2 * D_MODEL] = y[:, 1, :]

    return pl.pallas_call(
        tk,
        grid=((VOCAB + TCOLS - 1) // TCOLS,),
        in_specs=[pl.BlockSpec((D_MODEL, TCOLS), lambda i: (0, i))],
        out_specs=pl.BlockSpec((TCOLS // 2, 2 * D_MODEL), lambda i: (i, 0)),
        out_shape=jax.ShapeDtypeStruct((VOCAB // 2, 2 * D_MODEL), jnp.float32),
    )(table_t)


def kernel(x, table):
    batch, seq = x.shape
    xt_flat = x.T.reshape(batch * seq)
    table_lin = _detile_table(table.T).reshape(VOCAB, D_MODEL)
    out5 = _embed(xt_flat, table_lin)
    out = out5.transpose(2, 4, 0, 1, 3).reshape(batch, seq, D_MODEL)
    return out
